# Initial kernel scaffold; baseline (speedup 1.0000x reference)
import jax
import jax.numpy as jnp
from jax.experimental import pallas as pl


def kernel(x, edge_index, W_self0, W_neigh0, b0, W_self1, W_neigh1, b1):
    raise NotImplementedError("write your pallas kernel here")


# speedup vs baseline: 4.6449x; 4.6449x over previous
import functools

import jax
import jax.numpy as jnp
from jax import lax
from jax.experimental import pallas as pl
from jax.experimental.pallas import tpu as pltpu
from jax.experimental.pallas import tpu_sc as plsc

N_NODES = 10000
N_EDGES = 320000
D = 128

NC = 2
NS = 16
NW = NC * NS
NP = 10240
RT = NP // NS
EW = N_EDGES // NW
CH = 80
NCHUNK = EW // CH

_MESH = dict(core_axis_name="c", subcore_axis_name="s")


def _sc_sum_body(src_h, dst_h, xh, zrow, sum_out, srcv, dstv, rows, sem, sum_sh):
    c = lax.axis_index("c")
    s = lax.axis_index("s")
    wid = c * NS + s
    r0 = s * RT
    pltpu.sync_copy(zrow, sum_sh.at[pl.ds(r0, RT)])
    plsc.subcore_barrier()

    def body(i, carry):
        base = wid * EW + i * CH
        pltpu.sync_copy(src_h.at[pl.ds(base, CH)], srcv)
        pltpu.sync_copy(dst_h.at[pl.ds(base, CH)], dstv)
        pltpu.async_copy(xh.at[srcv], rows, sem).wait()
        pltpu.sync_copy(rows, sum_sh.at[dstv], add=True)
        return carry

    lax.fori_loop(0, NCHUNK, body, 0)
    plsc.subcore_barrier()
    pltpu.sync_copy(sum_sh.at[pl.ds(r0, RT)], sum_out.at[c, pl.ds(r0, RT)])


def _sc_deg_body(dst_h, ones_h, zrow, deg_out, dstv, rows, deg_sh):
    c = lax.axis_index("c")
    s = lax.axis_index("s")
    wid = c * NS + s
    r0 = s * RT
    pltpu.sync_copy(zrow, deg_sh.at[pl.ds(r0, RT)])
    pltpu.sync_copy(ones_h, rows)
    plsc.subcore_barrier()

    def body(i, carry):
        base = wid * EW + i * CH
        pltpu.sync_copy(dst_h.at[pl.ds(base, CH)], dstv)
        pltpu.sync_copy(rows, deg_sh.at[dstv], add=True)
        return carry

    lax.fori_loop(0, NCHUNK, body, 0)
    plsc.subcore_barrier()
    pltpu.sync_copy(deg_sh.at[pl.ds(r0, RT)], deg_out.at[c, pl.ds(r0, RT)])


def _make_sc_sum():
    return pl.kernel(
        _sc_sum_body,
        mesh=plsc.VectorSubcoreMesh(**_MESH),
        out_type=[jax.ShapeDtypeStruct((NC, NP, D), jnp.float32)],
        scratch_types=[
            pltpu.VMEM((CH,), jnp.int32),
            pltpu.VMEM((CH,), jnp.int32),
            pltpu.VMEM((CH, D), jnp.float32),
            pltpu.SemaphoreType.DMA,
            pltpu.VMEM_SHARED((NP, D), jnp.float32),
        ],
    )


def _make_sc_deg():
    return pl.kernel(
        _sc_deg_body,
        mesh=plsc.VectorSubcoreMesh(**_MESH),
        out_type=[jax.ShapeDtypeStruct((NC, NP, D), jnp.float32)],
        scratch_types=[
            pltpu.VMEM((CH,), jnp.int32),
            pltpu.VMEM((CH, D), jnp.float32),
            pltpu.VMEM_SHARED((NP, D), jnp.float32),
        ],
    )


def _tc_body0(x_ref, s0_ref, s1_ref, d0_ref, d1_ref, ws_ref, wn_ref, b_ref,
              o_ref, rdeg_ref):
    ssum = s0_ref[...] + s1_ref[...]
    deg = d0_ref[:, 0:1] + d1_ref[:, 0:1]
    rdeg = 1.0 / jnp.maximum(deg, 1.0)
    agg = ssum * rdeg
    acc = jnp.dot(x_ref[...], ws_ref[...], preferred_element_type=jnp.float32)
    acc = acc + jnp.dot(agg, wn_ref[...], preferred_element_type=jnp.float32)
    o_ref[...] = jnp.maximum(acc + b_ref[...], 0.0)
    rdeg_ref[...] = jnp.broadcast_to(rdeg, rdeg_ref.shape)


def _tc_body1(x_ref, s0_ref, s1_ref, rdeg_ref, ws_ref, wn_ref, b_ref, o_ref):
    agg = (s0_ref[...] + s1_ref[...]) * rdeg_ref[:, 0:1]
    acc = jnp.dot(x_ref[...], ws_ref[...], preferred_element_type=jnp.float32)
    acc = acc + jnp.dot(agg, wn_ref[...], preferred_element_type=jnp.float32)
    o_ref[...] = jnp.maximum(acc + b_ref[...], 0.0)


_BR = 2000


def _row_spec(w):
    return pl.BlockSpec((_BR, w), lambda i: (i, 0))


def _tc_layer0(x, s0, s1, d0, d1, ws, wn, b):
    w_spec = pl.BlockSpec((D, D), lambda i: (0, 0))
    return pl.pallas_call(
        _tc_body0,
        grid=(N_NODES // _BR,),
        in_specs=[_row_spec(D), _row_spec(D), _row_spec(D),
                  _row_spec(D), _row_spec(D),
                  w_spec, w_spec, pl.BlockSpec((1, D), lambda i: (0, 0))],
        out_specs=[_row_spec(D), _row_spec(16)],
        out_shape=[jax.ShapeDtypeStruct((N_NODES, D), jnp.float32),
                   jax.ShapeDtypeStruct((N_NODES, 16), jnp.float32)],
    )(x, s0, s1, d0, d1, ws, wn, b.reshape(1, D))


def _tc_layer1(h, s0, s1, rdeg, ws, wn, b):
    w_spec = pl.BlockSpec((D, D), lambda i: (0, 0))
    return pl.pallas_call(
        _tc_body1,
        grid=(N_NODES // _BR,),
        in_specs=[_row_spec(D), _row_spec(D), _row_spec(D), _row_spec(16),
                  w_spec, w_spec, pl.BlockSpec((1, D), lambda i: (0, 0))],
        out_specs=_row_spec(D),
        out_shape=jax.ShapeDtypeStruct((N_NODES, D), jnp.float32),
    )(h, s0, s1, rdeg, ws, wn, b.reshape(1, D))


def kernel(x, edge_index, W_self0, W_neigh0, b0, W_self1, W_neigh1, b1):
    ei = edge_index.astype(jnp.int32)
    src = ei[0]
    dst = ei[1]
    z = jnp.zeros((RT, D), jnp.float32)
    ones_rows = jnp.ones((CH, D), jnp.float32)

    sc_sum = _make_sc_sum()
    sc_deg = _make_sc_deg()

    (degs,) = sc_deg(dst, ones_rows, z)
    (sums0,) = sc_sum(src, dst, x, z)
    h1, rdeg = _tc_layer0(x, sums0[0, :N_NODES], sums0[1, :N_NODES],
                          degs[0, :N_NODES], degs[1, :N_NODES],
                          W_self0, W_neigh0, b0)
    (sums1,) = sc_sum(src, dst, h1, z)
    out = _tc_layer1(h1, sums1[0, :N_NODES], sums1[1, :N_NODES], rdeg,
                     W_self1, W_neigh1, b1)
    return out


# speedup vs baseline: 6.8005x; 1.4641x over previous
import functools

import jax
import jax.numpy as jnp
from jax import lax
from jax.experimental import pallas as pl
from jax.experimental.pallas import tpu as pltpu
from jax.experimental.pallas import tpu_sc as plsc

N_NODES = 10000
N_EDGES = 320000
D = 128

NC = 2
NS = 16
NW = NC * NS
NP = 10240
RT = NP // NS
EW = N_EDGES // NW
CH = 80
NCHUNK = EW // CH
NB = 4
NG = (NCHUNK - 1) // NB

_MESH = dict(core_axis_name="c", subcore_axis_name="s")


def _sc_sum_body(src_h, dst_h, xh, zrow, sum_out,
                 srcv, dstv, rows, gsem, ssem, sum_sh):
    c = lax.axis_index("c")
    s = lax.axis_index("s")
    wid = c * NS + s
    r0 = s * RT
    pltpu.sync_copy(zrow, sum_sh.at[pl.ds(r0, RT)])
    plsc.subcore_barrier()

    base = wid * EW
    pltpu.sync_copy(src_h.at[pl.ds(base, CH)], srcv.at[0])
    pltpu.sync_copy(dst_h.at[pl.ds(base, CH)], dstv.at[0])
    pltpu.async_copy(xh.at[srcv.at[0]], rows.at[0], gsem.at[0]).wait()
    pltpu.async_copy(rows.at[0], sum_sh.at[dstv.at[0]], ssem.at[0],
                     add=True).wait()

    def body(g, carry):
        base0 = wid * EW + CH + g * (NB * CH)
        gd = []
        for b in range(NB):
            base = base0 + b * CH
            pltpu.sync_copy(src_h.at[pl.ds(base, CH)], srcv.at[b])
            pltpu.sync_copy(dst_h.at[pl.ds(base, CH)], dstv.at[b])
            gd.append(pltpu.async_copy(xh.at[srcv.at[b]], rows.at[b],
                                       gsem.at[b]))
        sd = []
        for b in range(NB):
            gd[b].wait()
            sd.append(pltpu.async_copy(rows.at[b], sum_sh.at[dstv.at[b]],
                                       ssem.at[b], add=True))
        for b in range(NB):
            sd[b].wait()
        return carry

    lax.fori_loop(0, NG, body, 0)
    plsc.subcore_barrier()
    pltpu.sync_copy(sum_sh.at[pl.ds(r0, RT)], sum_out.at[c, pl.ds(r0, RT)])


def _sc_deg_body(dst_h, ones_h, zrow, deg_out, dstv, rows, ssem, deg_sh):
    c = lax.axis_index("c")
    s = lax.axis_index("s")
    wid = c * NS + s
    r0 = s * RT
    pltpu.sync_copy(zrow, deg_sh.at[pl.ds(r0, RT)])
    pltpu.sync_copy(ones_h, rows)
    plsc.subcore_barrier()

    pltpu.sync_copy(dst_h.at[pl.ds(wid * EW, CH)], dstv.at[0])
    pltpu.async_copy(rows, deg_sh.at[dstv.at[0]], ssem.at[0], add=True).wait()

    def body(g, carry):
        base0 = wid * EW + CH + g * (NB * CH)
        for b in range(NB):
            pltpu.sync_copy(dst_h.at[pl.ds(base0 + b * CH, CH)], dstv.at[b])
        sd = [pltpu.async_copy(rows, deg_sh.at[dstv.at[b]], ssem.at[b],
                               add=True) for b in range(NB)]
        for b in range(NB):
            sd[b].wait()
        return carry

    lax.fori_loop(0, NG, body, 0)
    plsc.subcore_barrier()
    pltpu.sync_copy(deg_sh.at[pl.ds(r0, RT)], deg_out.at[c, pl.ds(r0, RT)])


def _make_sc_sum():
    return pl.kernel(
        _sc_sum_body,
        mesh=plsc.VectorSubcoreMesh(**_MESH),
        out_type=[jax.ShapeDtypeStruct((NC, NP, D), jnp.float32)],
        scratch_types=[
            pltpu.VMEM((NB, CH), jnp.int32),
            pltpu.VMEM((NB, CH), jnp.int32),
            pltpu.VMEM((NB, CH, D), jnp.float32),
            pltpu.SemaphoreType.DMA((NB,)),
            pltpu.SemaphoreType.DMA((NB,)),
            pltpu.VMEM_SHARED((NP, D), jnp.float32),
        ],
    )


def _make_sc_deg():
    return pl.kernel(
        _sc_deg_body,
        mesh=plsc.VectorSubcoreMesh(**_MESH),
        out_type=[jax.ShapeDtypeStruct((NC, NP, D), jnp.float32)],
        scratch_types=[
            pltpu.VMEM((NB, CH), jnp.int32),
            pltpu.VMEM((CH, D), jnp.float32),
            pltpu.SemaphoreType.DMA((NB,)),
            pltpu.VMEM_SHARED((NP, D), jnp.float32),
        ],
    )


def _tc_body0(x_ref, s0_ref, s1_ref, d0_ref, d1_ref, ws_ref, wn_ref, b_ref,
              o_ref, rdeg_ref):
    ssum = s0_ref[...] + s1_ref[...]
    deg = d0_ref[:, 0:1] + d1_ref[:, 0:1]
    rdeg = 1.0 / jnp.maximum(deg, 1.0)
    agg = ssum * rdeg
    acc = jnp.dot(x_ref[...], ws_ref[...], preferred_element_type=jnp.float32)
    acc = acc + jnp.dot(agg, wn_ref[...], preferred_element_type=jnp.float32)
    o_ref[...] = jnp.maximum(acc + b_ref[...], 0.0)
    rdeg_ref[...] = jnp.broadcast_to(rdeg, rdeg_ref.shape)


def _tc_body1(x_ref, s0_ref, s1_ref, rdeg_ref, ws_ref, wn_ref, b_ref, o_ref):
    agg = (s0_ref[...] + s1_ref[...]) * rdeg_ref[:, 0:1]
    acc = jnp.dot(x_ref[...], ws_ref[...], preferred_element_type=jnp.float32)
    acc = acc + jnp.dot(agg, wn_ref[...], preferred_element_type=jnp.float32)
    o_ref[...] = jnp.maximum(acc + b_ref[...], 0.0)


_BR = 2000


def _row_spec(w):
    return pl.BlockSpec((_BR, w), lambda i: (i, 0))


def _tc_layer0(x, s0, s1, d0, d1, ws, wn, b):
    w_spec = pl.BlockSpec((D, D), lambda i: (0, 0))
    return pl.pallas_call(
        _tc_body0,
        grid=(N_NODES // _BR,),
        in_specs=[_row_spec(D), _row_spec(D), _row_spec(D),
                  _row_spec(D), _row_spec(D),
                  w_spec, w_spec, pl.BlockSpec((1, D), lambda i: (0, 0))],
        out_specs=[_row_spec(D), _row_spec(16)],
        out_shape=[jax.ShapeDtypeStruct((N_NODES, D), jnp.float32),
                   jax.ShapeDtypeStruct((N_NODES, 16), jnp.float32)],
    )(x, s0, s1, d0, d1, ws, wn, b.reshape(1, D))


def _tc_layer1(h, s0, s1, rdeg, ws, wn, b):
    w_spec = pl.BlockSpec((D, D), lambda i: (0, 0))
    return pl.pallas_call(
        _tc_body1,
        grid=(N_NODES // _BR,),
        in_specs=[_row_spec(D), _row_spec(D), _row_spec(D), _row_spec(16),
                  w_spec, w_spec, pl.BlockSpec((1, D), lambda i: (0, 0))],
        out_specs=_row_spec(D),
        out_shape=jax.ShapeDtypeStruct((N_NODES, D), jnp.float32),
    )(h, s0, s1, rdeg, ws, wn, b.reshape(1, D))


def kernel(x, edge_index, W_self0, W_neigh0, b0, W_self1, W_neigh1, b1):
    ei = edge_index.astype(jnp.int32)
    src = ei[0]
    dst = ei[1]
    z = jnp.zeros((RT, D), jnp.float32)
    ones_rows = jnp.ones((CH, D), jnp.float32)

    sc_sum = _make_sc_sum()
    sc_deg = _make_sc_deg()

    (degs,) = sc_deg(dst, ones_rows, z)
    (sums0,) = sc_sum(src, dst, x, z)
    h1, rdeg = _tc_layer0(x, sums0[0, :N_NODES], sums0[1, :N_NODES],
                          degs[0, :N_NODES], degs[1, :N_NODES],
                          W_self0, W_neigh0, b0)
    (sums1,) = sc_sum(src, dst, h1, z)
    out = _tc_layer1(h1, sums1[0, :N_NODES], sums1[1, :N_NODES], rdeg,
                     W_self1, W_neigh1, b1)
    return out
